# baseline (device time: 14713 ns/iter reference)
import jax
import jax.numpy as jnp
from jax import lax
from jax.experimental import pallas as pl
from jax.experimental.pallas import tpu as pltpu

N_DEV = 16

_SCORE_DOT = (((2,), (1,)), ((0,), (0,)))
_CTX_DOT = (((2,), (2,)), ((0,), (0,)))


def kernel(x, Wq, K_ext, V_ext, Wo):
    b, sq_loc, d_model = x.shape
    _, _, hq, dh = K_ext.shape

    kt = jnp.transpose(K_ext, (0, 2, 3, 1))
    vt = jnp.transpose(V_ext, (0, 2, 3, 1))

    def body(x_hbm, wq_hbm, kt_hbm, vt_hbm, wo_hbm, out_ref,
             x_v, wq_v, kt_v, vt_v, wo_v, kv_send, kv_win,
             in_sems, send_sems, recv_sems):
        my = lax.axis_index("i")
        left = lax.rem(my - 1 + N_DEV, N_DEV)
        right = lax.rem(my + 1, N_DEV)
        has_left = my != 0
        has_right = my != N_DEV - 1

        in_cps = [
            pltpu.make_async_copy(src, dst, in_sems.at[i])
            for i, (src, dst) in enumerate([
                (kt_hbm, kt_v), (vt_hbm, vt_v),
                (x_hbm, x_v), (wq_hbm, wq_v), (wo_hbm, wo_v),
            ])
        ]
        for cp in in_cps:
            cp.start()

        barrier_sem = pltpu.get_barrier_semaphore()

        @pl.when(has_right)
        def _():
            pl.semaphore_signal(
                barrier_sem, inc=1,
                device_id=(right,), device_id_type=pl.DeviceIdType.MESH,
            )

        @pl.when(has_left)
        def _():
            pl.semaphore_signal(
                barrier_sem, inc=1,
                device_id=(left,), device_id_type=pl.DeviceIdType.MESH,
            )

        is_edge = jnp.logical_or(my == 0, my == N_DEV - 1)

        @pl.when(is_edge)
        def _():
            pl.semaphore_wait(barrier_sem, 1)

        @pl.when(jnp.logical_not(is_edge))
        def _():
            pl.semaphore_wait(barrier_sem, 2)

        in_cps[0].wait()
        in_cps[1].wait()
        kv_send[0] = kt_v[...].astype(jnp.bfloat16)
        kv_send[1] = vt_v[...].astype(jnp.bfloat16)

        send_r = pltpu.make_async_remote_copy(
            src_ref=kv_send, dst_ref=kv_win.at[0],
            send_sem=send_sems.at[0], recv_sem=recv_sems.at[0],
            device_id=(right,), device_id_type=pl.DeviceIdType.MESH,
        )
        send_l = pltpu.make_async_remote_copy(
            src_ref=kv_send, dst_ref=kv_win.at[1],
            send_sem=send_sems.at[1], recv_sem=recv_sems.at[1],
            device_id=(left,), device_id_type=pl.DeviceIdType.MESH,
        )

        @pl.when(has_right)
        def _():
            send_r.start()

        @pl.when(has_left)
        def _():
            send_l.start()

        in_cps[2].wait()
        in_cps[3].wait()
        xf = x_v[...].reshape(b * sq_loc, d_model).astype(jnp.bfloat16)
        qp = jnp.dot(xf, wq_v[...].astype(jnp.bfloat16),
                     preferred_element_type=jnp.float32)
        q = (qp.reshape(b, sq_loc, hq, dh)
             .transpose(0, 2, 1, 3).astype(jnp.bfloat16))

        ctx_acc, den_acc = [], []
        for bb in range(b):
            s = lax.dot_general(q[bb], kv_send[0, bb], _SCORE_DOT,
                                preferred_element_type=jnp.float32)
            w = jnp.exp(s * 0.125)
            den_acc.append(jnp.sum(w, axis=-1, keepdims=True))
            ctx_acc.append(lax.dot_general(
                w.astype(jnp.bfloat16), kv_send[1, bb], _CTX_DOT,
                preferred_element_type=jnp.float32))

        qi = lax.broadcasted_iota(jnp.int32, (1, sq_loc, sq_loc), 1)
        kj = lax.broadcasted_iota(jnp.int32, (1, sq_loc, sq_loc), 2)
        for c_idx, gate, mask in ((0, has_left, qi <= kj),
                                  (1, has_right, kj <= qi)):

            @pl.when(gate)
            def _(recv=send_r if c_idx == 0 else send_l):
                recv.wait_recv()

            for bb in range(b):
                s = lax.dot_general(q[bb], kv_win[c_idx, 0, bb], _SCORE_DOT,
                                    preferred_element_type=jnp.float32)
                w = jnp.where(mask, jnp.exp(s * 0.125), 0.0)
                d_add = jnp.sum(w, axis=-1, keepdims=True)
                c_add = lax.dot_general(
                    w.astype(jnp.bfloat16), kv_win[c_idx, 1, bb], _CTX_DOT,
                    preferred_element_type=jnp.float32)
                den_acc[bb] += jnp.where(gate, d_add, 0.0)
                ctx_acc[bb] += jnp.where(gate, c_add, 0.0)

        ctx = jnp.stack([
            (ctx_acc[bb] / den_acc[bb]).transpose(1, 0, 2)
            for bb in range(b)
        ]).reshape(b * sq_loc, hq * dh)

        in_cps[4].wait()
        o = jnp.dot(ctx.astype(jnp.bfloat16), wo_v[...].astype(jnp.bfloat16),
                    preferred_element_type=jnp.float32)
        out_ref[...] = o.reshape(b, sq_loc, d_model)

        @pl.when(has_right)
        def _():
            send_r.wait_send()

        @pl.when(has_left)
        def _():
            send_l.wait_send()

    return pl.pallas_call(
        body,
        out_shape=jax.ShapeDtypeStruct((b, sq_loc, d_model), jnp.float32),
        in_specs=[pl.BlockSpec(memory_space=pl.ANY)] * 5,
        out_specs=pl.BlockSpec(memory_space=pltpu.VMEM),
        scratch_shapes=[
            pltpu.VMEM((b, sq_loc, d_model), jnp.float32),
            pltpu.VMEM((d_model, hq * dh), jnp.float32),
            pltpu.VMEM((b, hq, dh, sq_loc), jnp.float32),
            pltpu.VMEM((b, hq, dh, sq_loc), jnp.float32),
            pltpu.VMEM((hq * dh, d_model), jnp.float32),
            pltpu.VMEM((2, b, hq, dh, sq_loc), jnp.bfloat16),
            pltpu.VMEM((2, 2, b, hq, dh, sq_loc), jnp.bfloat16),
            pltpu.SemaphoreType.DMA((5,)),
            pltpu.SemaphoreType.DMA((2,)),
            pltpu.SemaphoreType.DMA((2,)),
        ],
        compiler_params=pltpu.CompilerParams(collective_id=0),
    )(x, Wq, kt, vt, Wo)


# device time: 12565 ns/iter; 1.1710x vs baseline; 1.1710x over previous
import jax
import jax.numpy as jnp
from jax import lax
from jax.experimental import pallas as pl
from jax.experimental.pallas import tpu as pltpu

N_DEV = 16

_SCORE_DOT = (((2,), (1,)), ((0,), (0,)))
_CTX_DOT = (((2,), (2,)), ((0,), (0,)))


def kernel(x, Wq, K_ext, V_ext, Wo):
    b, sq_loc, d_model = x.shape
    _, _, hq, dh = K_ext.shape

    kt = jnp.transpose(K_ext, (0, 2, 3, 1)).astype(jnp.bfloat16)
    vt = jnp.transpose(V_ext, (0, 2, 3, 1)).astype(jnp.bfloat16)
    xb = x.astype(jnp.bfloat16)
    wqb = Wq.astype(jnp.bfloat16)
    wob = Wo.astype(jnp.bfloat16)

    def body(x_ref, wq_ref, kt_ref, vt_ref, wo_ref, out_ref,
             kv_win, send_sems, recv_sems):
        my = lax.axis_index("i")
        left = lax.rem(my - 1 + N_DEV, N_DEV)
        right = lax.rem(my + 1, N_DEV)
        has_left = my != 0
        has_right = my != N_DEV - 1

        barrier_sem = pltpu.get_barrier_semaphore()

        @pl.when(has_right)
        def _():
            pl.semaphore_signal(
                barrier_sem, inc=1,
                device_id=(right,), device_id_type=pl.DeviceIdType.MESH,
            )

        @pl.when(has_left)
        def _():
            pl.semaphore_signal(
                barrier_sem, inc=1,
                device_id=(left,), device_id_type=pl.DeviceIdType.MESH,
            )

        is_edge = jnp.logical_or(my == 0, my == N_DEV - 1)

        @pl.when(is_edge)
        def _():
            pl.semaphore_wait(barrier_sem, 1)

        @pl.when(jnp.logical_not(is_edge))
        def _():
            pl.semaphore_wait(barrier_sem, 2)

        rdmas = []
        for i, (src, slot, kv, tgt, gate) in enumerate([
            (kt_ref, 0, 0, right, has_right),
            (vt_ref, 0, 1, right, has_right),
            (kt_ref, 1, 0, left, has_left),
            (vt_ref, 1, 1, left, has_left),
        ]):
            c = pltpu.make_async_remote_copy(
                src_ref=src, dst_ref=kv_win.at[slot, kv],
                send_sem=send_sems.at[i], recv_sem=recv_sems.at[i],
                device_id=(tgt,), device_id_type=pl.DeviceIdType.MESH,
            )

            @pl.when(gate)
            def _(c=c):
                c.start()

            rdmas.append(c)

        xf = x_ref[...].reshape(b * sq_loc, d_model)
        qp = jnp.dot(xf, wq_ref[...], preferred_element_type=jnp.float32)
        q = (qp.reshape(b, sq_loc, hq, dh)
             .transpose(0, 2, 1, 3).astype(jnp.bfloat16))

        ctx_acc, den_acc = [], []
        for bb in range(b):
            s = lax.dot_general(q[bb], kt_ref[bb], _SCORE_DOT,
                                preferred_element_type=jnp.float32)
            w = jnp.exp(s * 0.125)
            den_acc.append(jnp.sum(w, axis=-1, keepdims=True))
            ctx_acc.append(lax.dot_general(
                w.astype(jnp.bfloat16), vt_ref[bb], _CTX_DOT,
                preferred_element_type=jnp.float32))

        qi = lax.broadcasted_iota(jnp.int32, (1, sq_loc, sq_loc), 1)
        kj = lax.broadcasted_iota(jnp.int32, (1, sq_loc, sq_loc), 2)
        for slot, gate, mask in ((0, has_left, qi <= kj),
                                 (1, has_right, kj <= qi)):

            @pl.when(gate)
            def _(slot=slot):
                rdmas[2 * slot].wait_recv()
                rdmas[2 * slot + 1].wait_recv()

            for bb in range(b):
                s = lax.dot_general(q[bb], kv_win[slot, 0, bb], _SCORE_DOT,
                                    preferred_element_type=jnp.float32)
                w = jnp.where(mask, jnp.exp(s * 0.125), 0.0)
                d_add = jnp.sum(w, axis=-1, keepdims=True)
                c_add = lax.dot_general(
                    w.astype(jnp.bfloat16), kv_win[slot, 1, bb], _CTX_DOT,
                    preferred_element_type=jnp.float32)
                den_acc[bb] += jnp.where(gate, d_add, 0.0)
                ctx_acc[bb] += jnp.where(gate, c_add, 0.0)

        ctx = jnp.stack([
            (ctx_acc[bb] / den_acc[bb]).transpose(1, 0, 2)
            for bb in range(b)
        ]).reshape(b * sq_loc, hq * dh)

        o = jnp.dot(ctx.astype(jnp.bfloat16), wo_ref[...],
                    preferred_element_type=jnp.float32)
        out_ref[...] = o.reshape(b, sq_loc, d_model).astype(jnp.bfloat16)

        for i, gate in enumerate([has_right, has_right, has_left, has_left]):

            @pl.when(gate)
            def _(i=i):
                rdmas[i].wait_send()

    return pl.pallas_call(
        body,
        out_shape=jax.ShapeDtypeStruct((b, sq_loc, d_model), jnp.bfloat16),
        in_specs=[pl.BlockSpec(memory_space=pltpu.VMEM)] * 5,
        out_specs=pl.BlockSpec(memory_space=pltpu.VMEM),
        scratch_shapes=[
            pltpu.VMEM((2, 2, b, hq, dh, sq_loc), jnp.bfloat16),
            pltpu.SemaphoreType.DMA((4,)),
            pltpu.SemaphoreType.DMA((4,)),
        ],
        compiler_params=pltpu.CompilerParams(collective_id=0),
    )(xb, wqb, kt, vt, wob)
